# Initial kernel scaffold; baseline (speedup 1.0000x reference)
import jax
import jax.numpy as jnp
from jax.experimental import pallas as pl


def kernel(x, edge_index, edge_weight, W, b, gamma, beta):
    raise NotImplementedError("write your pallas kernel here")


# speedup vs baseline: 2.4587x; 2.4587x over previous
import functools

import jax
import jax.numpy as jnp
from jax import lax
from jax.experimental import pallas as pl
from jax.experimental.pallas import tpu as pltpu
from jax.experimental.pallas import tpu_sc as plsc

N_NODES = 10000
N_EDGES = 160000
IN_FEATS = 256
OUT_FEATS = 512
BN_EPS = 1e-5

HALF = IN_FEATS // 2
EDGE_CHUNK = 128
N_CHUNKS = N_EDGES // EDGE_CHUNK
N_SUBCORES = 16
CHUNK_ITERS = -(-N_CHUNKS // N_SUBCORES)
ROW_CHUNK = 80
N_ROW_CHUNKS = N_NODES // ROW_CHUNK
ROW_ITERS = -(-N_ROW_CHUNKS // N_SUBCORES)
LANES = 16


def _sc_body(x2_hbm, src_hbm, dst_hbm, w_hbm, out_hbm,
             srcv, dstv, w16v, gidxv, rows_v, drain_v, h_sh, sem):
  c = lax.axis_index("c")
  s = lax.axis_index("s")

  def _zrow(r, _):
    for jj in range(HALF // LANES):
      drain_v[r, pl.ds(jj * LANES, LANES)] = jnp.zeros((LANES,), jnp.float32)
    return _
  lax.fori_loop(0, ROW_CHUNK, _zrow, None)

  def _zchunk(k, _):
    ch = s + k * N_SUBCORES

    @pl.when(ch < N_ROW_CHUNKS)
    def _():
      pltpu.sync_copy(drain_v, h_sh.at[pl.ds(ch * ROW_CHUNK, ROW_CHUNK)])
    return _
  lax.fori_loop(0, ROW_ITERS, _zchunk, None)
  plsc.subcore_barrier()

  def _chunk(k, _):
    j = s + k * N_SUBCORES

    @pl.when(j < N_CHUNKS)
    def _():
      base = j * EDGE_CHUNK
      pltpu.sync_copy(src_hbm.at[pl.ds(base, EDGE_CHUNK)], srcv)
      pltpu.sync_copy(dst_hbm.at[pl.ds(base, EDGE_CHUNK)], dstv)
      pltpu.sync_copy(w_hbm.at[pl.ds(base, EDGE_CHUNK)], w16v)
      for t in range(EDGE_CHUNK // LANES):
        s16 = srcv[pl.ds(t * LANES, LANES)]
        gidxv[pl.ds(t * LANES, LANES)] = s16 + s16 + c
      pltpu.async_copy(x2_hbm.at[gidxv], rows_v, sem).wait()

      def _scale(e, _):
        wvec = w16v[e, :]
        for jj in range(HALF // LANES):
          sl = pl.ds(jj * LANES, LANES)
          rows_v[e, sl] = rows_v[e, sl] * wvec
        return _
      lax.fori_loop(0, EDGE_CHUNK, _scale, None)

      pltpu.sync_copy(rows_v, h_sh.at[dstv], add=True)
    return _
  lax.fori_loop(0, CHUNK_ITERS, _chunk, None)

  plsc.subcore_barrier()

  def _dchunk(k, _):
    ch = s + k * N_SUBCORES

    @pl.when(ch < N_ROW_CHUNKS)
    def _():
      pltpu.sync_copy(h_sh.at[pl.ds(ch * ROW_CHUNK, ROW_CHUNK)], drain_v)
      pltpu.sync_copy(
          drain_v, out_hbm.at[pl.ds(c * N_NODES + ch * ROW_CHUNK, ROW_CHUNK)])
    return _
  lax.fori_loop(0, ROW_ITERS, _dchunk, None)


_sc_scatter = functools.partial(
    pl.kernel,
    out_type=jax.ShapeDtypeStruct((2 * N_NODES, HALF), jnp.float32),
    mesh=plsc.VectorSubcoreMesh(core_axis_name="c", subcore_axis_name="s"),
    scratch_types=[
        pltpu.VMEM((EDGE_CHUNK,), jnp.int32),
        pltpu.VMEM((EDGE_CHUNK,), jnp.int32),
        pltpu.VMEM((EDGE_CHUNK, LANES), jnp.float32),
        pltpu.VMEM((EDGE_CHUNK,), jnp.int32),
        pltpu.VMEM((EDGE_CHUNK, HALF), jnp.float32),
        pltpu.VMEM((ROW_CHUNK, HALF), jnp.float32),
        pltpu.VMEM_SHARED((N_NODES, HALF), jnp.float32),
        pltpu.SemaphoreType.DMA,
    ],
)(_sc_body)


ROW_BLK = 1000
N_ROW_BLKS = N_NODES // ROW_BLK


def _mm_body(h_ref, w_ref, b_ref, y_ref, sum_ref, sq_ref):
  i = pl.program_id(0)
  g = pl.program_id(1)
  part = lax.dot_general(h_ref[...], w_ref[...], (((1,), (1,)), ((), ())),
                         preferred_element_type=jnp.float32)

  @pl.when(g == 0)
  def _():
    y_ref[...] = part

  @pl.when(g == 1)
  def _():
    y = jnp.maximum(y_ref[...] + part + b_ref[...], 0.0)
    y_ref[...] = y
    ps = jnp.sum(y, axis=0, keepdims=True)
    pq = jnp.sum(y * y, axis=0, keepdims=True)

    @pl.when(i == 0)
    def _():
      sum_ref[...] = ps
      sq_ref[...] = pq

    @pl.when(i > 0)
    def _():
      sum_ref[...] += ps
      sq_ref[...] += pq


def _bn_body(y_ref, sum_ref, sq_ref, g_ref, be_ref, o_ref):
  inv_n = 1.0 / N_NODES
  mean = sum_ref[...] * inv_n
  var = sq_ref[...] * inv_n - mean * mean
  scale = lax.rsqrt(var + BN_EPS) * g_ref[...]
  o_ref[...] = (y_ref[...] - mean) * scale + be_ref[...]


@jax.jit
def kernel(x, edge_index, edge_weight, W, b, gamma, beta):
  src = edge_index[0].astype(jnp.int32)
  dst = edge_index[1].astype(jnp.int32)
  x2 = x.reshape(2 * N_NODES, HALF)
  w16 = jnp.broadcast_to(edge_weight[:, None], (N_EDGES, LANES))

  h2 = _sc_scatter(x2, src, dst, w16)

  y, sums, sqs = pl.pallas_call(
      _mm_body,
      grid=(N_ROW_BLKS, 2),
      in_specs=[
          pl.BlockSpec((ROW_BLK, HALF), lambda i, g: (g * N_ROW_BLKS + i, 0)),
          pl.BlockSpec((OUT_FEATS, HALF), lambda i, g: (0, g)),
          pl.BlockSpec((1, OUT_FEATS), lambda i, g: (0, 0)),
      ],
      out_specs=[
          pl.BlockSpec((ROW_BLK, OUT_FEATS), lambda i, g: (i, 0)),
          pl.BlockSpec((1, OUT_FEATS), lambda i, g: (0, 0)),
          pl.BlockSpec((1, OUT_FEATS), lambda i, g: (0, 0)),
      ],
      out_shape=[
          jax.ShapeDtypeStruct((N_NODES, OUT_FEATS), jnp.float32),
          jax.ShapeDtypeStruct((1, OUT_FEATS), jnp.float32),
          jax.ShapeDtypeStruct((1, OUT_FEATS), jnp.float32),
      ],
  )(h2, W, b.reshape(1, OUT_FEATS))

  out = pl.pallas_call(
      _bn_body,
      grid=(N_ROW_BLKS,),
      in_specs=[
          pl.BlockSpec((ROW_BLK, OUT_FEATS), lambda i: (i, 0)),
          pl.BlockSpec((1, OUT_FEATS), lambda i: (0, 0)),
          pl.BlockSpec((1, OUT_FEATS), lambda i: (0, 0)),
          pl.BlockSpec((1, OUT_FEATS), lambda i: (0, 0)),
          pl.BlockSpec((1, OUT_FEATS), lambda i: (0, 0)),
      ],
      out_specs=pl.BlockSpec((ROW_BLK, OUT_FEATS), lambda i: (i, 0)),
      out_shape=jax.ShapeDtypeStruct((N_NODES, OUT_FEATS), jnp.float32),
  )(y, sums, sqs, gamma.reshape(1, OUT_FEATS), beta.reshape(1, OUT_FEATS))
  return out


# speedup vs baseline: 2.8721x; 1.1681x over previous
import functools

import jax
import jax.numpy as jnp
from jax import lax
from jax.experimental import pallas as pl
from jax.experimental.pallas import tpu as pltpu
from jax.experimental.pallas import tpu_sc as plsc

N_NODES = 10000
N_EDGES = 160000
IN_FEATS = 256
OUT_FEATS = 512
BN_EPS = 1e-5

HALF = IN_FEATS // 2
LANES = 16
N_SUBCORES = 16

CHUNK = 64
CHUNKS_PER_SUPER = 32
SUPER = CHUNK * CHUNKS_PER_SUPER
N_SUPER = -(-N_EDGES // SUPER)
E_PAD = N_SUPER * SUPER
SUPER_ITERS = -(-N_SUPER // N_SUBCORES)

ROW_CHUNK = 80
N_ROW_CHUNKS = N_NODES // ROW_CHUNK
ROW_ITERS = -(-N_ROW_CHUNKS // N_SUBCORES)


def _sc_body(x2_hbm, g0_hbm, g1_hbm, dst_hbm, w_hbm, out_hbm,
             gidxv, dstv0, dstv1, wv0, wv1, rows0, rows1, drain_v, h_sh,
             gsem, dsem0, dsem1):
  c = lax.axis_index("c")
  s = lax.axis_index("s")

  def _zrow(r, _):
    for jj in range(HALF // LANES):
      drain_v[r, pl.ds(jj * LANES, LANES)] = jnp.zeros((LANES,), jnp.float32)
    return _
  lax.fori_loop(0, ROW_CHUNK, _zrow, None)

  def _zchunk(k, _):
    ch = s + k * N_SUBCORES

    @pl.when(ch < N_ROW_CHUNKS)
    def _():
      pltpu.sync_copy(drain_v, h_sh.at[pl.ds(ch * ROW_CHUNK, ROW_CHUNK)])
    return _
  lax.fori_loop(0, ROW_ITERS, _zchunk, None)
  plsc.subcore_barrier()

  rows = (rows0, rows1)
  dstv = (dstv0, dstv1)
  wv = (wv0, wv1)
  dsem = (dsem0, dsem1)

  def _issue(base, kk, b):
    e0 = base + kk * CHUNK
    gd = pltpu.async_copy(
        x2_hbm.at[gidxv.at[pl.ds(kk * CHUNK, CHUNK)]], rows[b], dsem[b])
    wd = pltpu.async_copy(w_hbm.at[pl.ds(e0, CHUNK)], wv[b], dsem[b])
    dd = pltpu.async_copy(dst_hbm.at[pl.ds(e0, CHUNK)], dstv[b], dsem[b])
    return gd, wd, dd

  def _super(k, _):
    g = s + k * N_SUBCORES

    @pl.when(g < N_SUPER)
    def _():
      base = g * SUPER
      @pl.when(c == 0)
      def _():
        pltpu.sync_copy(g0_hbm.at[pl.ds(base, SUPER)], gidxv)

      @pl.when(c == 1)
      def _():
        pltpu.sync_copy(g1_hbm.at[pl.ds(base, SUPER)], gidxv)

      pend = _issue(base, 0, 0)
      for kk in range(CHUNKS_PER_SUPER):
        b = kk & 1
        if kk + 1 < CHUNKS_PER_SUPER:
          nxt = _issue(base, kk + 1, 1 - b)
        else:
          nxt = None
        for d in pend:
          d.wait()

        def _scale(e, _):
          wvec = wv[b][e, :]
          for jj in range(HALF // LANES):
            sl = pl.ds(jj * LANES, LANES)
            rows[b][e, sl] = rows[b][e, sl] * wvec
          return _
        lax.fori_loop(0, CHUNK, _scale, None)

        pltpu.sync_copy(rows[b], h_sh.at[dstv[b]], add=True)
        pend = nxt
    return _
  lax.fori_loop(0, SUPER_ITERS, _super, None)

  plsc.subcore_barrier()

  def _dchunk(k, _):
    ch = s + k * N_SUBCORES

    @pl.when(ch < N_ROW_CHUNKS)
    def _():
      pltpu.sync_copy(h_sh.at[pl.ds(ch * ROW_CHUNK, ROW_CHUNK)], drain_v)
      pltpu.sync_copy(
          drain_v, out_hbm.at[pl.ds(c * N_NODES + ch * ROW_CHUNK, ROW_CHUNK)])
    return _
  lax.fori_loop(0, ROW_ITERS, _dchunk, None)


_sc_scatter = functools.partial(
    pl.kernel,
    out_type=jax.ShapeDtypeStruct((2 * N_NODES, HALF), jnp.float32),
    mesh=plsc.VectorSubcoreMesh(core_axis_name="c", subcore_axis_name="s"),
    scratch_types=[
        pltpu.VMEM((SUPER,), jnp.int32),
        pltpu.VMEM((CHUNK,), jnp.int32),
        pltpu.VMEM((CHUNK,), jnp.int32),
        pltpu.VMEM((CHUNK, LANES), jnp.float32),
        pltpu.VMEM((CHUNK, LANES), jnp.float32),
        pltpu.VMEM((CHUNK, HALF), jnp.float32),
        pltpu.VMEM((CHUNK, HALF), jnp.float32),
        pltpu.VMEM((ROW_CHUNK, HALF), jnp.float32),
        pltpu.VMEM_SHARED((N_NODES, HALF), jnp.float32),
        pltpu.SemaphoreType.DMA,
        pltpu.SemaphoreType.DMA,
        pltpu.SemaphoreType.DMA,
    ],
)(_sc_body)


ROW_BLK = 1000
N_ROW_BLKS = N_NODES // ROW_BLK


def _mm_body(h_ref, w_ref, b_ref, y_ref, sum_ref, sq_ref):
  i = pl.program_id(0)
  g = pl.program_id(1)
  part = lax.dot_general(h_ref[...], w_ref[...], (((1,), (1,)), ((), ())),
                         preferred_element_type=jnp.float32)

  @pl.when(g == 0)
  def _():
    y_ref[...] = part

  @pl.when(g == 1)
  def _():
    y = jnp.maximum(y_ref[...] + part + b_ref[...], 0.0)
    y_ref[...] = y
    ps = jnp.sum(y, axis=0, keepdims=True)
    pq = jnp.sum(y * y, axis=0, keepdims=True)

    @pl.when(i == 0)
    def _():
      sum_ref[...] = ps
      sq_ref[...] = pq

    @pl.when(i > 0)
    def _():
      sum_ref[...] += ps
      sq_ref[...] += pq


def _bn_body(y_ref, sum_ref, sq_ref, g_ref, be_ref, o_ref):
  inv_n = 1.0 / N_NODES
  mean = sum_ref[...] * inv_n
  var = sq_ref[...] * inv_n - mean * mean
  scale = lax.rsqrt(var + BN_EPS) * g_ref[...]
  o_ref[...] = (y_ref[...] - mean) * scale + be_ref[...]


@jax.jit
def kernel(x, edge_index, edge_weight, W, b, gamma, beta):
  src = edge_index[0].astype(jnp.int32)
  dst = edge_index[1].astype(jnp.int32)
  x2 = x.reshape(2 * N_NODES, HALF)

  pad = E_PAD - N_EDGES
  g0 = jnp.pad(src * 2, (0, pad))
  g1 = jnp.pad(src * 2 + 1, (0, pad))
  dstp = jnp.pad(dst, (0, pad))
  w16 = jnp.pad(jnp.broadcast_to(edge_weight[:, None], (N_EDGES, LANES)),
                ((0, pad), (0, 0)))

  h2 = _sc_scatter(x2, g0, g1, dstp, w16)

  y, sums, sqs = pl.pallas_call(
      _mm_body,
      grid=(N_ROW_BLKS, 2),
      in_specs=[
          pl.BlockSpec((ROW_BLK, HALF), lambda i, g: (g * N_ROW_BLKS + i, 0)),
          pl.BlockSpec((OUT_FEATS, HALF), lambda i, g: (0, g)),
          pl.BlockSpec((1, OUT_FEATS), lambda i, g: (0, 0)),
      ],
      out_specs=[
          pl.BlockSpec((ROW_BLK, OUT_FEATS), lambda i, g: (i, 0)),
          pl.BlockSpec((1, OUT_FEATS), lambda i, g: (0, 0)),
          pl.BlockSpec((1, OUT_FEATS), lambda i, g: (0, 0)),
      ],
      out_shape=[
          jax.ShapeDtypeStruct((N_NODES, OUT_FEATS), jnp.float32),
          jax.ShapeDtypeStruct((1, OUT_FEATS), jnp.float32),
          jax.ShapeDtypeStruct((1, OUT_FEATS), jnp.float32),
      ],
  )(h2, W, b.reshape(1, OUT_FEATS))

  out = pl.pallas_call(
      _bn_body,
      grid=(N_ROW_BLKS,),
      in_specs=[
          pl.BlockSpec((ROW_BLK, OUT_FEATS), lambda i: (i, 0)),
          pl.BlockSpec((1, OUT_FEATS), lambda i: (0, 0)),
          pl.BlockSpec((1, OUT_FEATS), lambda i: (0, 0)),
          pl.BlockSpec((1, OUT_FEATS), lambda i: (0, 0)),
          pl.BlockSpec((1, OUT_FEATS), lambda i: (0, 0)),
      ],
      out_specs=pl.BlockSpec((ROW_BLK, OUT_FEATS), lambda i: (i, 0)),
      out_shape=jax.ShapeDtypeStruct((N_NODES, OUT_FEATS), jnp.float32),
  )(y, sums, sqs, gamma.reshape(1, OUT_FEATS), beta.reshape(1, OUT_FEATS))
  return out


# speedup vs baseline: 3.9817x; 1.3863x over previous
import functools

import jax
import jax.numpy as jnp
from jax import lax
from jax.experimental import pallas as pl
from jax.experimental.pallas import tpu as pltpu
from jax.experimental.pallas import tpu_sc as plsc

N_NODES = 10000
N_EDGES = 160000
IN_FEATS = 256
OUT_FEATS = 512
BN_EPS = 1e-5

HALF = IN_FEATS // 2
LANES = 16
N_SUBCORES = 16

CHUNK = 64
CHUNKS_PER_SUPER = 20
SUPER = CHUNK * CHUNKS_PER_SUPER
N_SUPER = N_EDGES // SUPER
SUPER_ITERS = -(-N_SUPER // N_SUBCORES)

ZROW_CHUNK = 40
N_ZROW_CHUNKS = N_NODES // ZROW_CHUNK
ZROW_ITERS = -(-N_ZROW_CHUNKS // N_SUBCORES)
DROW_CHUNK = 200
N_DROW_CHUNKS = N_NODES // DROW_CHUNK
DROW_ITERS = -(-N_DROW_CHUNKS // N_SUBCORES)


def _sc_body(x2_hbm, g0_hbm, g1_hbm, dst_hbm, w_hbm, out_hbm,
             gidxv, dstv0, dstv1, dstv2, wv0, wv1, wv2, rows0, rows1, rows2,
             h_sh, dsem0, dsem1, dsem2, ssem0, ssem1, ssem2):
  c = lax.axis_index("c")
  s = lax.axis_index("s")

  def _zrow(r, _):
    for jj in range(HALF // LANES):
      rows0[r, pl.ds(jj * LANES, LANES)] = jnp.zeros((LANES,), jnp.float32)
    return _
  lax.fori_loop(0, ZROW_CHUNK, _zrow, None)

  def _zchunk(k, _):
    ch = s + k * N_SUBCORES

    @pl.when(ch < N_ZROW_CHUNKS)
    def _():
      pltpu.sync_copy(rows0.at[pl.ds(0, ZROW_CHUNK)],
                      h_sh.at[pl.ds(ch * ZROW_CHUNK, ZROW_CHUNK)])
    return _
  lax.fori_loop(0, ZROW_ITERS, _zchunk, None)
  plsc.subcore_barrier()

  rows = (rows0, rows1, rows2)
  dstv = (dstv0, dstv1, dstv2)
  wv = (wv0, wv1, wv2)
  dsem = (dsem0, dsem1, dsem2)
  ssem = (ssem0, ssem1, ssem2)

  def _issue(base, kk, b):
    e0 = base + kk * CHUNK
    gd = pltpu.async_copy(
        x2_hbm.at[gidxv.at[pl.ds(kk * CHUNK, CHUNK)]], rows[b], dsem[b])
    wd = pltpu.async_copy(w_hbm.at[pl.ds(e0, CHUNK)], wv[b], dsem[b])
    dd = pltpu.async_copy(dst_hbm.at[pl.ds(e0, CHUNK)], dstv[b], dsem[b])
    return gd, wd, dd

  def _super(k, _):
    g = s + k * N_SUBCORES

    @pl.when(g < N_SUPER)
    def _():
      base = g * SUPER
      @pl.when(c == 0)
      def _():
        pltpu.sync_copy(g0_hbm.at[pl.ds(base, SUPER)], gidxv)

      @pl.when(c == 1)
      def _():
        pltpu.sync_copy(g1_hbm.at[pl.ds(base, SUPER)], gidxv)

      pend = _issue(base, 0, 0)
      scat = [None, None, None]
      for kk in range(CHUNKS_PER_SUPER):
        b = kk % 3
        if kk + 1 < CHUNKS_PER_SUPER:
          nb = (kk + 1) % 3
          if scat[nb] is not None:
            scat[nb].wait()
          nxt = _issue(base, kk + 1, nb)
        else:
          nxt = None
        for d in pend:
          d.wait()

        @plsc.parallel_loop(0, CHUNK, 1, unroll=2)
        def _scale(e, b=b):
          wvec = wv[b][e, :]
          for jj in range(HALF // LANES):
            sl = pl.ds(jj * LANES, LANES)
            rows[b][e, sl] = rows[b][e, sl] * wvec

        scat[b] = pltpu.async_copy(rows[b], h_sh.at[dstv[b]], ssem[b],
                                   add=True)
        pend = nxt
      for sd in scat:
        if sd is not None:
          sd.wait()
    return _
  lax.fori_loop(0, SUPER_ITERS, _super, None)

  plsc.subcore_barrier()

  def _dchunk(k, _):
    ch = s + k * N_SUBCORES

    @pl.when(ch < N_DROW_CHUNKS)
    def _():
      pltpu.sync_copy(
          h_sh.at[pl.ds(ch * DROW_CHUNK, DROW_CHUNK)],
          out_hbm.at[pl.ds(c * N_NODES + ch * DROW_CHUNK, DROW_CHUNK)])
    return _
  lax.fori_loop(0, DROW_ITERS, _dchunk, None)


_sc_scatter = functools.partial(
    pl.kernel,
    out_type=jax.ShapeDtypeStruct((2 * N_NODES, HALF), jnp.float32),
    mesh=plsc.VectorSubcoreMesh(core_axis_name="c", subcore_axis_name="s"),
    scratch_types=[
        pltpu.VMEM((SUPER,), jnp.int32),
        pltpu.VMEM((CHUNK,), jnp.int32),
        pltpu.VMEM((CHUNK,), jnp.int32),
        pltpu.VMEM((CHUNK,), jnp.int32),
        pltpu.VMEM((CHUNK, LANES), jnp.float32),
        pltpu.VMEM((CHUNK, LANES), jnp.float32),
        pltpu.VMEM((CHUNK, LANES), jnp.float32),
        pltpu.VMEM((CHUNK, HALF), jnp.float32),
        pltpu.VMEM((CHUNK, HALF), jnp.float32),
        pltpu.VMEM((CHUNK, HALF), jnp.float32),
        pltpu.VMEM_SHARED((N_NODES, HALF), jnp.float32),
        pltpu.SemaphoreType.DMA,
        pltpu.SemaphoreType.DMA,
        pltpu.SemaphoreType.DMA,
        pltpu.SemaphoreType.DMA,
        pltpu.SemaphoreType.DMA,
        pltpu.SemaphoreType.DMA,
    ],
)(_sc_body)


ROW_BLK = 1000
N_ROW_BLKS = N_NODES // ROW_BLK


def _mm_body(h_ref, w_ref, b_ref, y_ref, sum_ref, sq_ref):
  i = pl.program_id(0)
  g = pl.program_id(1)
  part = lax.dot_general(h_ref[...], w_ref[...], (((1,), (1,)), ((), ())),
                         preferred_element_type=jnp.float32)

  @pl.when(g == 0)
  def _():
    y_ref[...] = part

  @pl.when(g == 1)
  def _():
    y = jnp.maximum(y_ref[...] + part + b_ref[...], 0.0)
    y_ref[...] = y
    ps = jnp.sum(y, axis=0, keepdims=True)
    pq = jnp.sum(y * y, axis=0, keepdims=True)

    @pl.when(i == 0)
    def _():
      sum_ref[...] = ps
      sq_ref[...] = pq

    @pl.when(i > 0)
    def _():
      sum_ref[...] += ps
      sq_ref[...] += pq


def _bn_body(y_ref, sum_ref, sq_ref, g_ref, be_ref, o_ref):
  inv_n = 1.0 / N_NODES
  mean = sum_ref[...] * inv_n
  var = sq_ref[...] * inv_n - mean * mean
  scale = lax.rsqrt(var + BN_EPS) * g_ref[...]
  o_ref[...] = (y_ref[...] - mean) * scale + be_ref[...]


@jax.jit
def kernel(x, edge_index, edge_weight, W, b, gamma, beta):
  src = edge_index[0].astype(jnp.int32)
  dst = edge_index[1].astype(jnp.int32)
  x2 = x.reshape(2 * N_NODES, HALF)

  g0 = src * 2
  g1 = src * 2 + 1
  w16 = jnp.broadcast_to(edge_weight[:, None], (N_EDGES, LANES))

  h2 = _sc_scatter(x2, g0, g1, dst, w16)

  y, sums, sqs = pl.pallas_call(
      _mm_body,
      grid=(N_ROW_BLKS, 2),
      in_specs=[
          pl.BlockSpec((ROW_BLK, HALF), lambda i, g: (g * N_ROW_BLKS + i, 0)),
          pl.BlockSpec((OUT_FEATS, HALF), lambda i, g: (0, g)),
          pl.BlockSpec((1, OUT_FEATS), lambda i, g: (0, 0)),
      ],
      out_specs=[
          pl.BlockSpec((ROW_BLK, OUT_FEATS), lambda i, g: (i, 0)),
          pl.BlockSpec((1, OUT_FEATS), lambda i, g: (0, 0)),
          pl.BlockSpec((1, OUT_FEATS), lambda i, g: (0, 0)),
      ],
      out_shape=[
          jax.ShapeDtypeStruct((N_NODES, OUT_FEATS), jnp.float32),
          jax.ShapeDtypeStruct((1, OUT_FEATS), jnp.float32),
          jax.ShapeDtypeStruct((1, OUT_FEATS), jnp.float32),
      ],
  )(h2, W, b.reshape(1, OUT_FEATS))

  out = pl.pallas_call(
      _bn_body,
      grid=(N_ROW_BLKS,),
      in_specs=[
          pl.BlockSpec((ROW_BLK, OUT_FEATS), lambda i: (i, 0)),
          pl.BlockSpec((1, OUT_FEATS), lambda i: (0, 0)),
          pl.BlockSpec((1, OUT_FEATS), lambda i: (0, 0)),
          pl.BlockSpec((1, OUT_FEATS), lambda i: (0, 0)),
          pl.BlockSpec((1, OUT_FEATS), lambda i: (0, 0)),
      ],
      out_specs=pl.BlockSpec((ROW_BLK, OUT_FEATS), lambda i: (i, 0)),
      out_shape=jax.ShapeDtypeStruct((N_NODES, OUT_FEATS), jnp.float32),
  )(y, sums, sqs, gamma.reshape(1, OUT_FEATS), beta.reshape(1, OUT_FEATS))
  return out


# speedup vs baseline: 4.1053x; 1.0310x over previous
import functools

import jax
import jax.numpy as jnp
from jax import lax
from jax.experimental import pallas as pl
from jax.experimental.pallas import tpu as pltpu
from jax.experimental.pallas import tpu_sc as plsc

N_NODES = 10000
N_EDGES = 160000
IN_FEATS = 256
OUT_FEATS = 512
BN_EPS = 1e-5

HALF = IN_FEATS // 2
LANES = 16
N_SUBCORES = 16

CHUNK = 64
CHUNKS_PER_SUPER = 20
SUPER = CHUNK * CHUNKS_PER_SUPER
N_SUPER = N_EDGES // SUPER
SUPER_ITERS = -(-N_SUPER // N_SUBCORES)

ZROW_CHUNK = 40
N_ZROW_CHUNKS = N_NODES // ZROW_CHUNK
ZROW_ITERS = -(-N_ZROW_CHUNKS // N_SUBCORES)
DROW_CHUNK = 200
N_DROW_CHUNKS = N_NODES // DROW_CHUNK
DROW_ITERS = -(-N_DROW_CHUNKS // N_SUBCORES)


def _sc_body(x2_hbm, ei_hbm, w_hbm, out_hbm,
             gidxv, dstv0, dstv1, dstv2, wv0, wv1, wv2,
             rows0, rows1, rows2,
             h_sh, dsem0, dsem1, dsem2, ssem0, ssem1, ssem2):
  c = lax.axis_index("c")
  s = lax.axis_index("s")

  def _zrow(r, _):
    for jj in range(HALF // LANES):
      rows0[r, pl.ds(jj * LANES, LANES)] = jnp.zeros((LANES,), jnp.float32)
    return _
  lax.fori_loop(0, ZROW_CHUNK, _zrow, None)

  def _zchunk(k, _):
    ch = s + k * N_SUBCORES

    @pl.when(ch < N_ZROW_CHUNKS)
    def _():
      pltpu.sync_copy(rows0.at[pl.ds(0, ZROW_CHUNK)],
                      h_sh.at[pl.ds(ch * ZROW_CHUNK, ZROW_CHUNK)])
    return _
  lax.fori_loop(0, ZROW_ITERS, _zchunk, None)
  plsc.subcore_barrier()

  rows = (rows0, rows1, rows2)
  dstv = (dstv0, dstv1, dstv2)
  wv = (wv0, wv1, wv2)
  dsem = (dsem0, dsem1, dsem2)
  ssem = (ssem0, ssem1, ssem2)

  def _issue(base, kk, b):
    e0 = base + kk * CHUNK
    gd = pltpu.async_copy(
        x2_hbm.at[gidxv.at[pl.ds(kk * CHUNK, CHUNK)]], rows[b], dsem[b])
    wd = pltpu.async_copy(w_hbm.at[pl.ds(e0, CHUNK)], wv[b], dsem[b])
    dd = pltpu.async_copy(ei_hbm.at[1, pl.ds(e0, CHUNK)], dstv[b], dsem[b])
    return gd, wd, dd

  def _super(k, _):
    g = s + k * N_SUBCORES

    @pl.when(g < N_SUPER)
    def _():
      base = g * SUPER
      pltpu.sync_copy(ei_hbm.at[0, pl.ds(base, SUPER)], gidxv)

      @plsc.parallel_loop(0, SUPER // LANES, 1, unroll=4)
      def _gix(t):
        s16 = gidxv[pl.ds(t * LANES, LANES)]
        gidxv[pl.ds(t * LANES, LANES)] = s16 + s16 + c

      pend = _issue(base, 0, 0)
      scat = [None, None, None]
      for kk in range(CHUNKS_PER_SUPER):
        b = kk % 3
        if kk + 1 < CHUNKS_PER_SUPER:
          nb = (kk + 1) % 3
          if scat[nb] is not None:
            scat[nb].wait()
          nxt = _issue(base, kk + 1, nb)
        else:
          nxt = None
        for d in pend:
          d.wait()

        @plsc.parallel_loop(0, CHUNK, 1, unroll=4)
        def _scale(e, b=b):
          wvec = wv[b][e, :]
          for jj in range(HALF // LANES):
            sl = pl.ds(jj * LANES, LANES)
            rows[b][e, sl] = rows[b][e, sl] * wvec

        scat[b] = pltpu.async_copy(rows[b], h_sh.at[dstv[b]], ssem[b],
                                   add=True)
        pend = nxt
      for sd in scat:
        if sd is not None:
          sd.wait()
    return _
  lax.fori_loop(0, SUPER_ITERS, _super, None)

  plsc.subcore_barrier()

  def _dchunk(k, _):
    ch = s + k * N_SUBCORES

    @pl.when(ch < N_DROW_CHUNKS)
    def _():
      pltpu.sync_copy(
          h_sh.at[pl.ds(ch * DROW_CHUNK, DROW_CHUNK)],
          out_hbm.at[pl.ds(c * N_NODES + ch * DROW_CHUNK, DROW_CHUNK)])
    return _
  lax.fori_loop(0, DROW_ITERS, _dchunk, None)


_sc_scatter = functools.partial(
    pl.kernel,
    out_type=jax.ShapeDtypeStruct((2 * N_NODES, HALF), jnp.float32),
    mesh=plsc.VectorSubcoreMesh(core_axis_name="c", subcore_axis_name="s"),
    scratch_types=[
        pltpu.VMEM((SUPER,), jnp.int32),
        pltpu.VMEM((CHUNK,), jnp.int32),
        pltpu.VMEM((CHUNK,), jnp.int32),
        pltpu.VMEM((CHUNK,), jnp.int32),
        pltpu.VMEM((CHUNK, LANES), jnp.float32),
        pltpu.VMEM((CHUNK, LANES), jnp.float32),
        pltpu.VMEM((CHUNK, LANES), jnp.float32),
        pltpu.VMEM((CHUNK, HALF), jnp.float32),
        pltpu.VMEM((CHUNK, HALF), jnp.float32),
        pltpu.VMEM((CHUNK, HALF), jnp.float32),
        pltpu.VMEM_SHARED((N_NODES, HALF), jnp.float32),
        pltpu.SemaphoreType.DMA,
        pltpu.SemaphoreType.DMA,
        pltpu.SemaphoreType.DMA,
        pltpu.SemaphoreType.DMA,
        pltpu.SemaphoreType.DMA,
        pltpu.SemaphoreType.DMA,
    ],
)(_sc_body)


ROW_BLK = 1000
N_ROW_BLKS = N_NODES // ROW_BLK


def _mm_body(h_ref, w_ref, b_ref, y_ref, sum_ref, sq_ref):
  i = pl.program_id(0)
  g = pl.program_id(1)
  part = lax.dot_general(h_ref[...], w_ref[...], (((1,), (1,)), ((), ())),
                         preferred_element_type=jnp.float32)

  @pl.when(g == 0)
  def _():
    y_ref[...] = part

  @pl.when(g == 1)
  def _():
    y = jnp.maximum(y_ref[...] + part + b_ref[...], 0.0)
    y_ref[...] = y
    ps = jnp.sum(y, axis=0, keepdims=True)
    pq = jnp.sum(y * y, axis=0, keepdims=True)

    @pl.when(i == 0)
    def _():
      sum_ref[...] = ps
      sq_ref[...] = pq

    @pl.when(i > 0)
    def _():
      sum_ref[...] += ps
      sq_ref[...] += pq


def _bn_body(y_ref, sum_ref, sq_ref, g_ref, be_ref, o_ref):
  inv_n = 1.0 / N_NODES
  mean = sum_ref[...] * inv_n
  var = sq_ref[...] * inv_n - mean * mean
  scale = lax.rsqrt(var + BN_EPS) * g_ref[...]
  o_ref[...] = (y_ref[...] - mean) * scale + be_ref[...]


@jax.jit
def kernel(x, edge_index, edge_weight, W, b, gamma, beta):
  x2 = x.reshape(2 * N_NODES, HALF)

  w16 = jnp.broadcast_to(edge_weight[:, None], (N_EDGES, LANES))

  h2 = _sc_scatter(x2, edge_index.astype(jnp.int32), w16)

  y, sums, sqs = pl.pallas_call(
      _mm_body,
      grid=(N_ROW_BLKS, 2),
      in_specs=[
          pl.BlockSpec((ROW_BLK, HALF), lambda i, g: (g * N_ROW_BLKS + i, 0)),
          pl.BlockSpec((OUT_FEATS, HALF), lambda i, g: (0, g)),
          pl.BlockSpec((1, OUT_FEATS), lambda i, g: (0, 0)),
      ],
      out_specs=[
          pl.BlockSpec((ROW_BLK, OUT_FEATS), lambda i, g: (i, 0)),
          pl.BlockSpec((1, OUT_FEATS), lambda i, g: (0, 0)),
          pl.BlockSpec((1, OUT_FEATS), lambda i, g: (0, 0)),
      ],
      out_shape=[
          jax.ShapeDtypeStruct((N_NODES, OUT_FEATS), jnp.float32),
          jax.ShapeDtypeStruct((1, OUT_FEATS), jnp.float32),
          jax.ShapeDtypeStruct((1, OUT_FEATS), jnp.float32),
      ],
  )(h2, W, b.reshape(1, OUT_FEATS))

  out = pl.pallas_call(
      _bn_body,
      grid=(N_ROW_BLKS,),
      in_specs=[
          pl.BlockSpec((ROW_BLK, OUT_FEATS), lambda i: (i, 0)),
          pl.BlockSpec((1, OUT_FEATS), lambda i: (0, 0)),
          pl.BlockSpec((1, OUT_FEATS), lambda i: (0, 0)),
          pl.BlockSpec((1, OUT_FEATS), lambda i: (0, 0)),
          pl.BlockSpec((1, OUT_FEATS), lambda i: (0, 0)),
      ],
      out_specs=pl.BlockSpec((ROW_BLK, OUT_FEATS), lambda i: (i, 0)),
      out_shape=jax.ShapeDtypeStruct((N_NODES, OUT_FEATS), jnp.float32),
  )(y, sums, sqs, gamma.reshape(1, OUT_FEATS), beta.reshape(1, OUT_FEATS))
  return out
